# baseline (speedup 1.0000x reference)
import jax
import jax.numpy as jnp
import numpy as np
from jax import lax
from jax.experimental import pallas as pl
from jax.experimental.pallas import tpu as pltpu
from jax.experimental.pallas import tpu_sc as plsc

CH = [64, 128, 256]
SHAPES = [(32, 32, 8), (16, 16, 8), (8, 8, 4)]
EMBED, NH, NL, NP = 128, 4, 3, 4
HD = EMBED // NH
NQ = sum(d * h * w for d, h, w in SHAPES)
LEVEL_START = [0, 8192, 10240]
NG = NH * NL * NP
TQ = 256
NT = NQ // TQ

_INTERP = False


def _a1_body(x10, x20, x11, x21, x12, x22,
             qw0, vw0, qw1, vw1, qw2, vw2,
             qb0, vb0, qb1, vb1, qb2, vb2, q_ref, tab_ref):
    cdims = (((0,), (0,)), ((), ()))
    s = 0
    for x1, x2, qw, vw, qb, vb, (d, h, w) in (
        (x10, x20, qw0, vw0, qb0, vb0, SHAPES[0]),
        (x11, x21, qw1, vw1, qb1, vb1, SHAPES[1]),
        (x12, x22, qw2, vw2, qb2, vb2, SHAPES[2]),
    ):
        n = d * h * w
        q_ref[s:s + n, :] = lax.dot_general(
            x1[...], qw[...], cdims, preferred_element_type=jnp.float32) + qb[...]
        v = lax.dot_general(
            x2[...], vw[...], cdims, preferred_element_type=jnp.float32) + vb[...]
        for hh in range(NH):
            tab_ref[hh, s:s + n, :] = v[:, hh * HD:(hh + 1) * HD].astype(jnp.bfloat16)
        s += n


def _stage_a1(x1s, x2s, qws, vws, qbs, vbs):
    return pl.pallas_call(
        _a1_body,
        out_shape=(jax.ShapeDtypeStruct((NQ, EMBED), jnp.float32),
                   jax.ShapeDtypeStruct((NH, NQ, HD), jnp.bfloat16)),
        interpret=_INTERP,
    )(x1s[0], x2s[0], x1s[1], x2s[1], x1s[2], x2s[2],
      qws[0], vws[0], qws[1], vws[1], qws[2], vws[2],
      qbs[0], vbs[0], qbs[1], vbs[1], qbs[2], vbs[2])


def _a2_body(q_ref, refq, aww, awb, sowr, sowa, sowe, sob, consts_f, consts_i,
             idx_ref, w_ref):
    q = q_ref[...]
    logits = jnp.dot(q, aww[...], preferred_element_type=jnp.float32) + awb[...]
    parts = []
    for h in range(NH):
        sl = logits[:, h * 12:(h + 1) * 12]
        m = jnp.max(sl, axis=1, keepdims=True)
        e = jnp.exp(sl - m)
        parts.append(e / jnp.sum(e, axis=1, keepdims=True))
    attn = jnp.concatenate(parts, axis=1)

    sd = consts_f[0, 0:1, :]
    sh = consts_f[0, 1:2, :]
    sw = consts_f[0, 2:3, :]
    di = consts_i[0, 0:1, :]
    hi = consts_i[0, 1:2, :]
    wi = consts_i[0, 2:3, :]
    base = consts_i[0, 3:4, :]

    cr = (refq[:, 0:1] * sd + sob[0, 0:1, :]
          + jnp.dot(q, sowr[...], preferred_element_type=jnp.float32) - 0.5)
    ca = (refq[:, 1:2] * sh + sob[0, 1:2, :]
          + jnp.dot(q, sowa[...], preferred_element_type=jnp.float32) - 0.5)
    ce = (refq[:, 2:3] * sw + sob[0, 2:3, :]
          + jnp.dot(q, sowe[...], preferred_element_type=jnp.float32) - 0.5)
    r0f = jnp.floor(cr)
    a0f = jnp.floor(ca)
    e0f = jnp.floor(ce)
    wr = cr - r0f
    wa = ca - a0f
    we = ce - e0f
    r0 = r0f.astype(jnp.int32)
    a0 = a0f.astype(jnp.int32)
    e0 = e0f.astype(jnp.int32)

    def axis_terms(x0, frac, bound, stride, extra):
        w_lo = jnp.where((x0 >= 0) & (x0 < bound), 1.0 - frac, 0.0)
        w_hi = jnp.where((x0 >= -1) & (x0 < bound - 1), frac, 0.0)
        i_lo = jnp.clip(x0, 0, bound - 1) * stride + extra
        i_hi = jnp.clip(x0 + 1, 0, bound - 1) * stride + extra
        return (w_lo, w_hi), (i_lo, i_hi)

    zero = jnp.zeros_like(base)
    wrv, rterm = axis_terms(r0, wr, di, hi * wi, base)
    wav, aterm = axis_terms(a0, wa, hi, wi, zero)
    wev, eterm = axis_terms(e0, we, wi, 1, zero)
    wrwa = [[wrv[0] * wav[0], wrv[0] * wav[1]],
            [wrv[1] * wav[0], wrv[1] * wav[1]]]
    wea = [wev[0] * attn, wev[1] * attn]

    idx_parts = [[] for _ in range(NH)]
    w_parts = [[] for _ in range(NH)]
    for c8 in range(8):
        dr, da, de = (c8 >> 2) & 1, (c8 >> 1) & 1, c8 & 1
        idx_c = rterm[dr] + aterm[da] + eterm[de]
        w_c = wrwa[dr][da] * wea[de]
        for h in range(NH):
            idx_parts[h].append(idx_c[:, h * 12:(h + 1) * 12])
            w_parts[h].append(w_c[:, h * 12:(h + 1) * 12])
    for h in range(NH):
        idx_ref[h] = jnp.concatenate(idx_parts[h], axis=1)
        w_ref[h] = jnp.concatenate(w_parts[h], axis=1)


def _stage_a2(q, refq, aww, awb, sowr, sowa, sowe, sob, consts_f, consts_i):
    return pl.pallas_call(
        _a2_body,
        grid=(NT,),
        in_specs=[
            pl.BlockSpec((TQ, EMBED), lambda i: (i, 0)),
            pl.BlockSpec((TQ, 3), lambda i: (i, 0)),
            pl.BlockSpec((EMBED, NG), lambda i: (0, 0)),
            pl.BlockSpec((1, NG), lambda i: (0, 0)),
            pl.BlockSpec((EMBED, NG), lambda i: (0, 0)),
            pl.BlockSpec((EMBED, NG), lambda i: (0, 0)),
            pl.BlockSpec((EMBED, NG), lambda i: (0, 0)),
            pl.BlockSpec((1, 3, NG), lambda i: (0, 0, 0)),
            pl.BlockSpec((1, 3, NG), lambda i: (0, 0, 0)),
            pl.BlockSpec((1, 4, NG), lambda i: (0, 0, 0)),
        ],
        out_specs=[
            pl.BlockSpec((NH, TQ, 96), lambda i: (0, i, 0)),
            pl.BlockSpec((NH, TQ, 96), lambda i: (0, i, 0)),
        ],
        out_shape=(jax.ShapeDtypeStruct((NH, NQ, 96), jnp.int32),
                   jax.ShapeDtypeStruct((NH, NQ, 96), jnp.float32)),
        interpret=_INTERP,
    )(q, refq, aww, awb, sowr, sowa, sowe, sob, consts_f, consts_i)


QB = 32
ROWS = 96 * QB
QPW = NQ // 8
NCHUNK = QPW // QB


def _sc_body(idx_hbm, w_hbm, table_hbm, acc_hbm,
             idx_v0, idx_v1, w_v0, w_v1, rows_v0, rows_v1, acc_v0, acc_v1,
             sem0, sem1, ssemi0, ssemi1, ssemw0, ssemw1, wbsem0, wbsem1):
    nc = 2
    wid = lax.axis_index("s") * nc + lax.axis_index("c")
    h = wid % NH
    qb = wid // NH
    q0 = qb * QPW

    idx_vs = (idx_v0, idx_v1)
    w_vs = (w_v0, w_v1)
    rows_vs = (rows_v0, rows_v1)
    acc_vs = (acc_v0, acc_v1)
    sems = (sem0, sem1)
    ssemis = (ssemi0, ssemi1)
    ssemws = (ssemw0, ssemw1)
    wbsems = (wbsem0, wbsem1)

    def stage_idx_start(it, b):
        base = (q0 + it * QB) * 96
        pltpu.async_copy(idx_hbm.at[h, pl.ds(base, ROWS)], idx_vs[b], ssemis[b])

    def stage_w_start(it, b):
        base = (q0 + it * QB) * 96
        pltpu.async_copy(w_hbm.at[h, pl.ds(base, ROWS)], w_vs[b], ssemws[b])

    def stage_wait(it, b):
        base = (q0 + it * QB) * 96
        pltpu.make_async_copy(idx_hbm.at[h, pl.ds(base, ROWS)], idx_vs[b],
                              ssemis[b]).wait()
        pltpu.make_async_copy(w_hbm.at[h, pl.ds(base, ROWS)], w_vs[b],
                              ssemws[b]).wait()

    def gather_start(b):
        pltpu.async_copy(table_hbm.at[idx_vs[b]], rows_vs[b], sems[b])

    def gather_wait(b):
        pltpu.make_async_copy(table_hbm.at[idx_vs[b]], rows_vs[b],
                              sems[b]).wait()

    def wb_start(it, b):
        pltpu.async_copy(acc_vs[b], acc_hbm.at[h, pl.ds(q0 + it * QB, QB)],
                         wbsems[b])

    def wb_wait(it, b):
        pltpu.make_async_copy(acc_vs[b], acc_hbm.at[h, pl.ds(q0 + it * QB, QB)],
                              wbsems[b]).wait()

    def compute(b):
        rows_v = rows_vs[b]
        w_v = w_vs[b]
        acc_v = acc_vs[b]

        @plsc.parallel_loop(0, QB, unroll=2)
        def per_q(q):
            a0 = [jnp.zeros((16,), jnp.float32) for _ in range(4)]
            a1 = [jnp.zeros((16,), jnp.float32) for _ in range(4)]
            for k in range(6):
                wv = w_v[pl.ds(q * 96 + k * 16, 16)]
                for j in range(16):
                    wspl = jnp.take_along_axis(
                        wv, jnp.full((16,), j, jnp.int32), axis=0,
                        mode="promise_in_bounds")
                    ev, od = plsc.unpack(rows_v[q * 96 + k * 16 + j],
                                         format=plsc.PackFormat.INTERLEAVED)
                    c = j % 4
                    a0[c] = a0[c] + wspl * ev
                    a1[c] = a1[c] + wspl * od
            acc_v[q, 0:16] = (a0[0] + a0[1]) + (a0[2] + a0[3])
            acc_v[q, 16:32] = (a1[0] + a1[1]) + (a1[2] + a1[3])

    def step(it, b):
        @pl.when(it + 1 < NCHUNK)
        def _g_next():
            stage_wait(it + 1, 1 - b)
            gather_start(1 - b)

        gather_wait(b)

        @pl.when(it + 2 < NCHUNK)
        def _s_next_idx():
            stage_idx_start(it + 2, b)

        @pl.when(it >= 2)
        def _wb_prev():
            wb_wait(it - 2, b)

        compute(b)

        @pl.when(it + 2 < NCHUNK)
        def _s_next_w():
            stage_w_start(it + 2, b)

        wb_start(it, b)

    stage_idx_start(0, 0)
    stage_w_start(0, 0)
    stage_wait(0, 0)
    gather_start(0)
    stage_idx_start(1, 1)
    stage_w_start(1, 1)

    def pair(i, _):
        step(i * 2, 0)
        step(i * 2 + 1, 1)
        return _

    lax.fori_loop(0, NCHUNK // 2, pair, None, unroll=False)
    step(NCHUNK - 1, (NCHUNK - 1) % 2)
    wb_wait(NCHUNK - 2, NCHUNK % 2)
    wb_wait(NCHUNK - 1, (NCHUNK - 1) % 2)


def _stage_b(idx_flat, w_flat, table):
    mesh = plsc.VectorSubcoreMesh(core_axis_name="c", subcore_axis_name="s")
    f = pl.kernel(
        _sc_body,
        out_type=jax.ShapeDtypeStruct((NH, NQ, HD), jnp.float32),
        mesh=mesh,
        scratch_types=[
            pltpu.VMEM((ROWS,), jnp.int32),
            pltpu.VMEM((ROWS,), jnp.int32),
            pltpu.VMEM((ROWS,), jnp.float32),
            pltpu.VMEM((ROWS,), jnp.float32),
            pltpu.VMEM((ROWS, HD), jnp.bfloat16),
            pltpu.VMEM((ROWS, HD), jnp.bfloat16),
            pltpu.VMEM((QB, HD), jnp.float32),
            pltpu.VMEM((QB, HD), jnp.float32),
            pltpu.SemaphoreType.DMA,
            pltpu.SemaphoreType.DMA,
            pltpu.SemaphoreType.DMA,
            pltpu.SemaphoreType.DMA,
            pltpu.SemaphoreType.DMA,
            pltpu.SemaphoreType.DMA,
            pltpu.SemaphoreType.DMA,
            pltpu.SemaphoreType.DMA,
        ],
        compiler_params=pltpu.CompilerParams(needs_layout_passes=False,
                                             use_tc_tiling_on_sc=False),
    )
    return f(idx_flat, w_flat, table)


def _c_body(acc, q, ow, ob, out_ref):
    acc_t = jnp.concatenate([acc[h] for h in range(NH)], axis=1)
    out_ref[...] = (jnp.dot(acc_t, ow[...], preferred_element_type=jnp.float32)
                    + ob[...] + q[...])


def _stage_c(acc, query, out_w, out_b):
    return pl.pallas_call(
        _c_body,
        grid=(NT,),
        in_specs=[
            pl.BlockSpec((NH, TQ, HD), lambda i: (0, i, 0)),
            pl.BlockSpec((TQ, EMBED), lambda i: (i, 0)),
            pl.BlockSpec((EMBED, EMBED), lambda i: (0, 0)),
            pl.BlockSpec((1, EMBED), lambda i: (0, 0)),
        ],
        out_specs=pl.BlockSpec((TQ, EMBED), lambda i: (i, 0)),
        out_shape=jax.ShapeDtypeStruct((NQ, EMBED), jnp.float32),
        interpret=_INTERP,
    )(acc, query, out_w, out_b)


def _ref_points(flows):
    refs = []
    for flow, (r, a, e) in zip(flows, SHAPES):
        n = r * a * e
        f = (flow / jnp.array([r, a, e], jnp.float32)).reshape(3, n)
        gr = (jnp.arange(r, dtype=jnp.float32) + 0.5) / r
        ga = (jnp.arange(a, dtype=jnp.float32) + 0.5) / a
        ge = (jnp.arange(e, dtype=jnp.float32) + 0.5) / e
        grid = jnp.stack(jnp.meshgrid(gr, ga, ge, indexing='ij'), axis=-1)
        refs.append(grid.reshape(n, 3) + f.T)
    return jnp.concatenate(refs, axis=0)


def _consts():
    sd = np.zeros((NG,), np.float32)
    sh = np.zeros((NG,), np.float32)
    sw = np.zeros((NG,), np.float32)
    di = np.zeros((NG,), np.int32)
    hi = np.zeros((NG,), np.int32)
    wi = np.zeros((NG,), np.int32)
    base = np.zeros((NG,), np.int32)
    for h in range(NH):
        for l in range(NL):
            d, hh, w = SHAPES[l]
            for p in range(NP):
                g = h * 12 + l * 4 + p
                sd[g] = d
                sh[g] = hh
                sw[g] = w
                di[g] = d
                hi[g] = hh
                wi[g] = w
                base[g] = h * NQ + LEVEL_START[l]
    consts_f = jnp.asarray(np.stack([sd, sh, sw])[None])
    consts_i = jnp.asarray(np.stack([di, hi, wi, base])[None])
    return consts_f, consts_i


def kernel(x1_msf_0, x1_msf_1, x1_msf_2, x2_msf_0, x2_msf_1, x2_msf_2,
           flow_0, flow_1, flow_2, qw0, qb0, vw0, vb0, qw1, qb1, vw1, vb1,
           qw2, qb2, vw2, vb2, so_w, so_b, aw_w, aw_b, out_w, out_b):
    x1s = [x1_msf_0.reshape(CH[0], -1), x1_msf_1.reshape(CH[1], -1),
           x1_msf_2.reshape(CH[2], -1)]
    x2s = [x2_msf_0.reshape(CH[0], -1), x2_msf_1.reshape(CH[1], -1),
           x2_msf_2.reshape(CH[2], -1)]
    qbs = [qb0[None, :], qb1[None, :], qb2[None, :]]
    vbs = [vb0[None, :], vb1[None, :], vb2[None, :]]

    query, table = _stage_a1(x1s, x2s, [qw0, qw1, qw2], [vw0, vw1, vw2],
                             qbs, vbs)

    refq = _ref_points([flow_0, flow_1, flow_2])
    sowr = so_w[:, 0::3]
    sowa = so_w[:, 1::3]
    sowe = so_w[:, 2::3]
    sob = jnp.stack([so_b[0::3], so_b[1::3], so_b[2::3]])[None]
    consts_f, consts_i = _consts()
    idx, w = _stage_a2(query, refq, aw_w, aw_b[None, :], sowr, sowa, sowe,
                       sob, consts_f, consts_i)

    acc = _stage_b(idx.reshape(NH, NQ * 96), w.reshape(NH, NQ * 96),
                   table.reshape(NH * NQ, HD))

    perm = np.concatenate([np.arange(0, HD, 2), np.arange(1, HD, 2)])
    row_perm = np.concatenate([h * HD + perm for h in range(NH)])
    out = _stage_c(acc, query, out_w[row_perm, :], out_b[None, :])
    return out[None]


# speedup vs baseline: 1.0407x; 1.0407x over previous
import jax
import jax.numpy as jnp
import numpy as np
from jax import lax
from jax.experimental import pallas as pl
from jax.experimental.pallas import tpu as pltpu
from jax.experimental.pallas import tpu_sc as plsc

CH = [64, 128, 256]
SHAPES = [(32, 32, 8), (16, 16, 8), (8, 8, 4)]
EMBED, NH, NL, NP = 128, 4, 3, 4
HD = EMBED // NH
NQ = sum(d * h * w for d, h, w in SHAPES)
LEVEL_START = [0, 8192, 10240]
NG = NH * NL * NP
TQ = 656
NT = NQ // TQ

_INTERP = False


def _a1_body(x10, x20, x11, x21, x12, x22,
             qw0, vw0, qw1, vw1, qw2, vw2,
             qb0, vb0, qb1, vb1, qb2, vb2, q_ref, tab_ref):
    cdims = (((0,), (0,)), ((), ()))
    s = 0
    for x1, x2, qw, vw, qb, vb, (d, h, w) in (
        (x10, x20, qw0, vw0, qb0, vb0, SHAPES[0]),
        (x11, x21, qw1, vw1, qb1, vb1, SHAPES[1]),
        (x12, x22, qw2, vw2, qb2, vb2, SHAPES[2]),
    ):
        n = d * h * w
        q_ref[s:s + n, :] = lax.dot_general(
            x1[...], qw[...], cdims, preferred_element_type=jnp.float32) + qb[...]
        v = lax.dot_general(
            x2[...], vw[...], cdims, preferred_element_type=jnp.float32) + vb[...]
        for hh in range(NH):
            tab_ref[hh, s:s + n, :] = v[:, hh * HD:(hh + 1) * HD].astype(jnp.bfloat16)
        s += n


def _stage_a1(x1s, x2s, qws, vws, qbs, vbs):
    return pl.pallas_call(
        _a1_body,
        out_shape=(jax.ShapeDtypeStruct((NQ, EMBED), jnp.float32),
                   jax.ShapeDtypeStruct((NH, NQ, HD), jnp.bfloat16)),
        interpret=_INTERP,
    )(x1s[0], x2s[0], x1s[1], x2s[1], x1s[2], x2s[2],
      qws[0], vws[0], qws[1], vws[1], qws[2], vws[2],
      qbs[0], vbs[0], qbs[1], vbs[1], qbs[2], vbs[2])


def _a2_body(q_ref, refq, aww, awb, sowr, sowa, sowe, sob, consts_f, consts_i,
             idx_ref, w_ref):
    q = q_ref[...]
    logits = jnp.dot(q, aww[...], preferred_element_type=jnp.float32) + awb[...]
    parts = []
    for h in range(NH):
        sl = logits[:, h * 12:(h + 1) * 12]
        m = jnp.max(sl, axis=1, keepdims=True)
        e = jnp.exp(sl - m)
        parts.append(e / jnp.sum(e, axis=1, keepdims=True))
    attn = jnp.concatenate(parts, axis=1)

    sd = consts_f[0, 0:1, :]
    sh = consts_f[0, 1:2, :]
    sw = consts_f[0, 2:3, :]
    di = consts_i[0, 0:1, :]
    hi = consts_i[0, 1:2, :]
    wi = consts_i[0, 2:3, :]
    base = consts_i[0, 3:4, :]

    cr = (refq[:, 0:1] * sd + sob[0, 0:1, :]
          + jnp.dot(q, sowr[...], preferred_element_type=jnp.float32) - 0.5)
    ca = (refq[:, 1:2] * sh + sob[0, 1:2, :]
          + jnp.dot(q, sowa[...], preferred_element_type=jnp.float32) - 0.5)
    ce = (refq[:, 2:3] * sw + sob[0, 2:3, :]
          + jnp.dot(q, sowe[...], preferred_element_type=jnp.float32) - 0.5)
    r0f = jnp.floor(cr)
    a0f = jnp.floor(ca)
    e0f = jnp.floor(ce)
    wr = cr - r0f
    wa = ca - a0f
    we = ce - e0f
    r0 = r0f.astype(jnp.int32)
    a0 = a0f.astype(jnp.int32)
    e0 = e0f.astype(jnp.int32)

    def axis_terms(x0, frac, bound, stride, extra):
        w_lo = jnp.where((x0 >= 0) & (x0 < bound), 1.0 - frac, 0.0)
        w_hi = jnp.where((x0 >= -1) & (x0 < bound - 1), frac, 0.0)
        i_lo = jnp.clip(x0, 0, bound - 1) * stride + extra
        i_hi = jnp.clip(x0 + 1, 0, bound - 1) * stride + extra
        return (w_lo, w_hi), (i_lo, i_hi)

    zero = jnp.zeros_like(base)
    wrv, rterm = axis_terms(r0, wr, di, hi * wi, base)
    wav, aterm = axis_terms(a0, wa, hi, wi, zero)
    wev, eterm = axis_terms(e0, we, wi, 1, zero)
    wrwa = [[wrv[0] * wav[0], wrv[0] * wav[1]],
            [wrv[1] * wav[0], wrv[1] * wav[1]]]
    wea = [wev[0] * attn, wev[1] * attn]

    idx_parts = [[] for _ in range(NH)]
    w_parts = [[] for _ in range(NH)]
    for c8 in range(8):
        dr, da, de = (c8 >> 2) & 1, (c8 >> 1) & 1, c8 & 1
        idx_c = rterm[dr] + aterm[da] + eterm[de]
        w_c = wrwa[dr][da] * wea[de]
        for h in range(NH):
            idx_parts[h].append(idx_c[:, h * 12:(h + 1) * 12])
            w_parts[h].append(w_c[:, h * 12:(h + 1) * 12])
    for h in range(NH):
        idx_ref[h] = jnp.concatenate(idx_parts[h], axis=1)
        w_ref[h] = jnp.concatenate(w_parts[h], axis=1)


def _stage_a2(q, refq, aww, awb, sowr, sowa, sowe, sob, consts_f, consts_i):
    return pl.pallas_call(
        _a2_body,
        grid=(NT,),
        in_specs=[
            pl.BlockSpec((TQ, EMBED), lambda i: (i, 0)),
            pl.BlockSpec((TQ, 3), lambda i: (i, 0)),
            pl.BlockSpec((EMBED, NG), lambda i: (0, 0)),
            pl.BlockSpec((1, NG), lambda i: (0, 0)),
            pl.BlockSpec((EMBED, NG), lambda i: (0, 0)),
            pl.BlockSpec((EMBED, NG), lambda i: (0, 0)),
            pl.BlockSpec((EMBED, NG), lambda i: (0, 0)),
            pl.BlockSpec((1, 3, NG), lambda i: (0, 0, 0)),
            pl.BlockSpec((1, 3, NG), lambda i: (0, 0, 0)),
            pl.BlockSpec((1, 4, NG), lambda i: (0, 0, 0)),
        ],
        out_specs=[
            pl.BlockSpec((NH, TQ, 96), lambda i: (0, i, 0)),
            pl.BlockSpec((NH, TQ, 96), lambda i: (0, i, 0)),
        ],
        out_shape=(jax.ShapeDtypeStruct((NH, NQ, 96), jnp.int32),
                   jax.ShapeDtypeStruct((NH, NQ, 96), jnp.float32)),
        interpret=_INTERP,
    )(q, refq, aww, awb, sowr, sowa, sowe, sob, consts_f, consts_i)


QB = 32
ROWS = 96 * QB
QPW = NQ // 8
NCHUNK = QPW // QB


def _sc_body(idx_hbm, w_hbm, table_hbm, acc_hbm,
             idx_v0, idx_v1, w_v0, w_v1, rows_v0, rows_v1, acc_v0, acc_v1,
             sem0, sem1, ssemi0, ssemi1, ssemw0, ssemw1, wbsem0, wbsem1):
    nc = 2
    wid = lax.axis_index("s") * nc + lax.axis_index("c")
    h = wid % NH
    qb = wid // NH
    q0 = qb * QPW

    idx_vs = (idx_v0, idx_v1)
    w_vs = (w_v0, w_v1)
    rows_vs = (rows_v0, rows_v1)
    acc_vs = (acc_v0, acc_v1)
    sems = (sem0, sem1)
    ssemis = (ssemi0, ssemi1)
    ssemws = (ssemw0, ssemw1)
    wbsems = (wbsem0, wbsem1)

    def stage_idx_start(it, b):
        base = (q0 + it * QB) * 96
        pltpu.async_copy(idx_hbm.at[h, pl.ds(base, ROWS)], idx_vs[b], ssemis[b])

    def stage_w_start(it, b):
        base = (q0 + it * QB) * 96
        pltpu.async_copy(w_hbm.at[h, pl.ds(base, ROWS)], w_vs[b], ssemws[b])

    def stage_wait(it, b):
        base = (q0 + it * QB) * 96
        pltpu.make_async_copy(idx_hbm.at[h, pl.ds(base, ROWS)], idx_vs[b],
                              ssemis[b]).wait()
        pltpu.make_async_copy(w_hbm.at[h, pl.ds(base, ROWS)], w_vs[b],
                              ssemws[b]).wait()

    def gather_start(b):
        pltpu.async_copy(table_hbm.at[idx_vs[b]], rows_vs[b], sems[b])

    def gather_wait(b):
        pltpu.make_async_copy(table_hbm.at[idx_vs[b]], rows_vs[b],
                              sems[b]).wait()

    def wb_start(it, b):
        pltpu.async_copy(acc_vs[b], acc_hbm.at[h, pl.ds(q0 + it * QB, QB)],
                         wbsems[b])

    def wb_wait(it, b):
        pltpu.make_async_copy(acc_vs[b], acc_hbm.at[h, pl.ds(q0 + it * QB, QB)],
                              wbsems[b]).wait()

    def compute(b):
        rows_v = rows_vs[b]
        w_v = w_vs[b]
        acc_v = acc_vs[b]

        @plsc.parallel_loop(0, QB, unroll=2)
        def per_q(q):
            a0 = [jnp.zeros((16,), jnp.float32) for _ in range(4)]
            a1 = [jnp.zeros((16,), jnp.float32) for _ in range(4)]
            for k in range(6):
                wv = w_v[pl.ds(q * 96 + k * 16, 16)]
                for j in range(16):
                    wspl = jnp.take_along_axis(
                        wv, jnp.full((16,), j, jnp.int32), axis=0,
                        mode="promise_in_bounds")
                    ev, od = plsc.unpack(rows_v[q * 96 + k * 16 + j],
                                         format=plsc.PackFormat.INTERLEAVED)
                    c = j % 4
                    a0[c] = a0[c] + wspl * ev
                    a1[c] = a1[c] + wspl * od
            acc_v[q, 0:16] = (a0[0] + a0[1]) + (a0[2] + a0[3])
            acc_v[q, 16:32] = (a1[0] + a1[1]) + (a1[2] + a1[3])

    def step(it, b):
        @pl.when(it + 1 < NCHUNK)
        def _g_next():
            stage_wait(it + 1, 1 - b)
            gather_start(1 - b)

        gather_wait(b)

        @pl.when(it + 2 < NCHUNK)
        def _s_next_idx():
            stage_idx_start(it + 2, b)

        @pl.when(it >= 2)
        def _wb_prev():
            wb_wait(it - 2, b)

        compute(b)

        @pl.when(it + 2 < NCHUNK)
        def _s_next_w():
            stage_w_start(it + 2, b)

        wb_start(it, b)

    stage_idx_start(0, 0)
    stage_w_start(0, 0)
    stage_wait(0, 0)
    gather_start(0)
    stage_idx_start(1, 1)
    stage_w_start(1, 1)

    def pair(i, _):
        step(i * 2, 0)
        step(i * 2 + 1, 1)
        return _

    lax.fori_loop(0, NCHUNK // 2, pair, None, unroll=False)
    step(NCHUNK - 1, (NCHUNK - 1) % 2)
    wb_wait(NCHUNK - 2, NCHUNK % 2)
    wb_wait(NCHUNK - 1, (NCHUNK - 1) % 2)


def _stage_b(idx_flat, w_flat, table):
    mesh = plsc.VectorSubcoreMesh(core_axis_name="c", subcore_axis_name="s")
    f = pl.kernel(
        _sc_body,
        out_type=jax.ShapeDtypeStruct((NH, NQ, HD), jnp.float32),
        mesh=mesh,
        scratch_types=[
            pltpu.VMEM((ROWS,), jnp.int32),
            pltpu.VMEM((ROWS,), jnp.int32),
            pltpu.VMEM((ROWS,), jnp.float32),
            pltpu.VMEM((ROWS,), jnp.float32),
            pltpu.VMEM((ROWS, HD), jnp.bfloat16),
            pltpu.VMEM((ROWS, HD), jnp.bfloat16),
            pltpu.VMEM((QB, HD), jnp.float32),
            pltpu.VMEM((QB, HD), jnp.float32),
            pltpu.SemaphoreType.DMA,
            pltpu.SemaphoreType.DMA,
            pltpu.SemaphoreType.DMA,
            pltpu.SemaphoreType.DMA,
            pltpu.SemaphoreType.DMA,
            pltpu.SemaphoreType.DMA,
            pltpu.SemaphoreType.DMA,
            pltpu.SemaphoreType.DMA,
        ],
        compiler_params=pltpu.CompilerParams(needs_layout_passes=False,
                                             use_tc_tiling_on_sc=False),
    )
    return f(idx_flat, w_flat, table)


def _c_body(acc, q, ow, ob, out_ref):
    acc_t = jnp.concatenate([acc[h] for h in range(NH)], axis=1)
    out_ref[...] = (jnp.dot(acc_t, ow[...], preferred_element_type=jnp.float32)
                    + ob[...] + q[...])


def _stage_c(acc, query, out_w, out_b):
    return pl.pallas_call(
        _c_body,
        grid=(NT,),
        in_specs=[
            pl.BlockSpec((NH, TQ, HD), lambda i: (0, i, 0)),
            pl.BlockSpec((TQ, EMBED), lambda i: (i, 0)),
            pl.BlockSpec((EMBED, EMBED), lambda i: (0, 0)),
            pl.BlockSpec((1, EMBED), lambda i: (0, 0)),
        ],
        out_specs=pl.BlockSpec((TQ, EMBED), lambda i: (i, 0)),
        out_shape=jax.ShapeDtypeStruct((NQ, EMBED), jnp.float32),
        interpret=_INTERP,
    )(acc, query, out_w, out_b)


def _ref_points(flows):
    refs = []
    for flow, (r, a, e) in zip(flows, SHAPES):
        n = r * a * e
        f = (flow / jnp.array([r, a, e], jnp.float32)).reshape(3, n)
        gr = (jnp.arange(r, dtype=jnp.float32) + 0.5) / r
        ga = (jnp.arange(a, dtype=jnp.float32) + 0.5) / a
        ge = (jnp.arange(e, dtype=jnp.float32) + 0.5) / e
        grid = jnp.stack(jnp.meshgrid(gr, ga, ge, indexing='ij'), axis=-1)
        refs.append(grid.reshape(n, 3) + f.T)
    return jnp.concatenate(refs, axis=0)


def _consts():
    sd = np.zeros((NG,), np.float32)
    sh = np.zeros((NG,), np.float32)
    sw = np.zeros((NG,), np.float32)
    di = np.zeros((NG,), np.int32)
    hi = np.zeros((NG,), np.int32)
    wi = np.zeros((NG,), np.int32)
    base = np.zeros((NG,), np.int32)
    for h in range(NH):
        for l in range(NL):
            d, hh, w = SHAPES[l]
            for p in range(NP):
                g = h * 12 + l * 4 + p
                sd[g] = d
                sh[g] = hh
                sw[g] = w
                di[g] = d
                hi[g] = hh
                wi[g] = w
                base[g] = h * NQ + LEVEL_START[l]
    consts_f = jnp.asarray(np.stack([sd, sh, sw])[None])
    consts_i = jnp.asarray(np.stack([di, hi, wi, base])[None])
    return consts_f, consts_i


def kernel(x1_msf_0, x1_msf_1, x1_msf_2, x2_msf_0, x2_msf_1, x2_msf_2,
           flow_0, flow_1, flow_2, qw0, qb0, vw0, vb0, qw1, qb1, vw1, vb1,
           qw2, qb2, vw2, vb2, so_w, so_b, aw_w, aw_b, out_w, out_b):
    x1s = [x1_msf_0.reshape(CH[0], -1), x1_msf_1.reshape(CH[1], -1),
           x1_msf_2.reshape(CH[2], -1)]
    x2s = [x2_msf_0.reshape(CH[0], -1), x2_msf_1.reshape(CH[1], -1),
           x2_msf_2.reshape(CH[2], -1)]
    qbs = [qb0[None, :], qb1[None, :], qb2[None, :]]
    vbs = [vb0[None, :], vb1[None, :], vb2[None, :]]

    query, table = _stage_a1(x1s, x2s, [qw0, qw1, qw2], [vw0, vw1, vw2],
                             qbs, vbs)

    refq = _ref_points([flow_0, flow_1, flow_2])
    sowr = so_w[:, 0::3]
    sowa = so_w[:, 1::3]
    sowe = so_w[:, 2::3]
    sob = jnp.stack([so_b[0::3], so_b[1::3], so_b[2::3]])[None]
    consts_f, consts_i = _consts()
    idx, w = _stage_a2(query, refq, aw_w, aw_b[None, :], sowr, sowa, sowe,
                       sob, consts_f, consts_i)

    acc = _stage_b(idx.reshape(NH, NQ * 96), w.reshape(NH, NQ * 96),
                   table.reshape(NH * NQ, HD))

    perm = np.concatenate([np.arange(0, HD, 2), np.arange(1, HD, 2)])
    row_perm = np.concatenate([h * HD + perm for h in range(NH)])
    out = _stage_c(acc, query, out_w[row_perm, :], out_b[None, :])
    return out[None]
